# initial kernel scaffold (unmeasured)
import jax
import jax.numpy as jnp
from jax import lax
from jax.experimental import pallas as pl
from jax.experimental.pallas import tpu as pltpu

N_DEV = 4


def kernel(x, w_mat):
    m_glob, k_per = x.shape
    k_per_w, n = w_mat.shape
    m_per = m_glob // N_DEV

    def body(x_ref, w_ref, out_ref, bufs, send_sems, recv_sems):
        p = lax.axis_index("i")
        left = lax.rem(p + N_DEV - 1, N_DEV)
        right = lax.rem(p + 1, N_DEV)

        barrier_sem = pltpu.get_barrier_semaphore()
        for nbr in (left, right):
            pl.semaphore_signal(
                barrier_sem, inc=1,
                device_id=(nbr,), device_id_type=pl.DeviceIdType.MESH,
            )
        pl.semaphore_wait(barrier_sem, 2)

        def partial(c):
            xs = x_ref[pl.ds(c * m_per, m_per), :]
            return jnp.dot(xs, w_ref[:, :], preferred_element_type=jnp.float32)

        bufs[0, :, :] = partial(lax.rem(p + N_DEV - 1, N_DEV))

        for s in range(N_DEV - 1):
            rdma = pltpu.make_async_remote_copy(
                src_ref=bufs.at[s],
                dst_ref=bufs.at[s + 1],
                send_sem=send_sems.at[s],
                recv_sem=recv_sems.at[s],
                device_id=(right,),
                device_id_type=pl.DeviceIdType.MESH,
            )
            rdma.start()
            rdma.wait()
            c = lax.rem(p + 2 * N_DEV - 2 - s, N_DEV)
            if s < N_DEV - 2:
                bufs[s + 1, :, :] = bufs[s + 1, :, :] + partial(c)
            else:
                y = bufs[s + 1, :, :] + partial(c)
                out_ref[:, :] = jax.nn.gelu(y, approximate=True)

    return pl.pallas_call(
        body,
        out_shape=jax.ShapeDtypeStruct((m_per, n), jnp.float32),
        in_specs=[
            pl.BlockSpec(memory_space=pltpu.VMEM),
            pl.BlockSpec(memory_space=pltpu.VMEM),
        ],
        out_specs=pl.BlockSpec(memory_space=pltpu.VMEM),
        scratch_shapes=[
            pltpu.VMEM((N_DEV, m_per, n), jnp.float32),
            pltpu.SemaphoreType.DMA((N_DEV - 1,)),
            pltpu.SemaphoreType.DMA((N_DEV - 1,)),
        ],
        compiler_params=pltpu.CompilerParams(collective_id=0),
    )(x, w_mat)


# baseline (device time: 317522 ns/iter reference)
import jax
import jax.numpy as jnp
from jax import lax
from jax.experimental import pallas as pl
from jax.experimental.pallas import tpu as pltpu

N_DEV = 4


def kernel(x, w_mat):
    m_glob, k_per = x.shape
    k_per_w, n = w_mat.shape
    m_per = m_glob // N_DEV

    def body(x_ref, w_ref, out_ref, bufs, send_sems, recv_sems):
        p = lax.axis_index("i")
        left = lax.rem(p + N_DEV - 1, N_DEV)
        right = lax.rem(p + 1, N_DEV)

        barrier_sem = pltpu.get_barrier_semaphore()
        for nbr in (left, right):
            pl.semaphore_signal(
                barrier_sem, inc=1,
                device_id=(nbr,), device_id_type=pl.DeviceIdType.MESH,
            )
        pl.semaphore_wait(barrier_sem, 2)

        def partial(c):
            xs = x_ref[pl.ds(c * m_per, m_per), :]
            return jnp.dot(xs, w_ref[:, :], preferred_element_type=jnp.float32)

        bufs[0, :, :] = partial(lax.rem(p + N_DEV - 1, N_DEV))

        for s in range(N_DEV - 1):
            rdma = pltpu.make_async_remote_copy(
                src_ref=bufs.at[s % 3],
                dst_ref=bufs.at[(s + 1) % 3],
                send_sem=send_sems.at[s],
                recv_sem=recv_sems.at[s],
                device_id=(right,),
                device_id_type=pl.DeviceIdType.MESH,
            )
            rdma.start()
            rdma.wait()
            c = lax.rem(p + 2 * N_DEV - 2 - s, N_DEV)
            if s < N_DEV - 2:
                r = (s + 1) % 3
                bufs[r, :, :] = bufs[r, :, :] + partial(c)
            else:
                y = bufs[(s + 1) % 3, :, :] + partial(c)
                out_ref[:, :] = jax.nn.gelu(y, approximate=True)

    return pl.pallas_call(
        body,
        out_shape=jax.ShapeDtypeStruct((m_per, n), jnp.float32),
        in_specs=[
            pl.BlockSpec(memory_space=pltpu.VMEM),
            pl.BlockSpec(memory_space=pltpu.VMEM),
        ],
        out_specs=pl.BlockSpec(memory_space=pltpu.VMEM),
        scratch_shapes=[
            pltpu.VMEM((3, m_per, n), jnp.float32),
            pltpu.SemaphoreType.DMA((N_DEV - 1,)),
            pltpu.SemaphoreType.DMA((N_DEV - 1,)),
        ],
        compiler_params=pltpu.CompilerParams(
            collective_id=0,
            vmem_limit_bytes=100 * 1024 * 1024,
        ),
    )(x, w_mat)


# device time: 169298 ns/iter; 1.8755x vs baseline; 1.8755x over previous
import jax
import jax.numpy as jnp
from jax import lax
from jax.experimental import pallas as pl
from jax.experimental.pallas import tpu as pltpu

N_DEV = 4


def kernel(x, w_mat):
    m_glob, k_per = x.shape
    _, n = w_mat.shape
    m_per = m_glob // N_DEV
    n_half = n // 2

    def body(x_ref, w_ref, out_ref,
             cw_bufs, ccw_bufs,
             cw_send_sems, cw_recv_sems, ccw_send_sems, ccw_recv_sems):
        p = lax.axis_index("i")
        left = lax.rem(p + N_DEV - 1, N_DEV)
        right = lax.rem(p + 1, N_DEV)

        barrier_sem = pltpu.get_barrier_semaphore()
        for nbr in (left, right):
            pl.semaphore_signal(
                barrier_sem, inc=1,
                device_id=(nbr,), device_id_type=pl.DeviceIdType.MESH,
            )
        pl.semaphore_wait(barrier_sem, 2)

        def partial(c, half):
            xs = x_ref[pl.ds(c * m_per, m_per), :]
            ws = w_ref[:, half * n_half:(half + 1) * n_half]
            return jnp.dot(xs, ws, preferred_element_type=jnp.float32)

        cw_bufs[0, :, :] = partial(lax.rem(p + N_DEV - 1, N_DEV), 0)
        ccw_bufs[0, :, :] = partial(lax.rem(p + 1, N_DEV), 1)

        for s in range(N_DEV - 1):
            snd, rcv = s % 2, (s + 1) % 2
            cw_rdma = pltpu.make_async_remote_copy(
                src_ref=cw_bufs.at[snd],
                dst_ref=cw_bufs.at[rcv],
                send_sem=cw_send_sems.at[s],
                recv_sem=cw_recv_sems.at[s],
                device_id=(right,),
                device_id_type=pl.DeviceIdType.MESH,
            )
            ccw_rdma = pltpu.make_async_remote_copy(
                src_ref=ccw_bufs.at[snd],
                dst_ref=ccw_bufs.at[rcv],
                send_sem=ccw_send_sems.at[s],
                recv_sem=ccw_recv_sems.at[s],
                device_id=(left,),
                device_id_type=pl.DeviceIdType.MESH,
            )
            cw_rdma.start()
            ccw_rdma.start()

            c_cw = lax.rem(p + 2 * N_DEV - 2 - s, N_DEV)
            c_ccw = lax.rem(p + 2 + s, N_DEV)
            t_cw = partial(c_cw, 0)
            t_ccw = partial(c_ccw, 1)

            cw_rdma.wait()
            ccw_rdma.wait()

            if s < N_DEV - 2:
                cw_bufs[rcv, :, :] = cw_bufs[rcv, :, :] + t_cw
                ccw_bufs[rcv, :, :] = ccw_bufs[rcv, :, :] + t_ccw
            else:
                out_ref[:, :n_half] = jax.nn.gelu(
                    cw_bufs[rcv, :, :] + t_cw, approximate=True)
                out_ref[:, n_half:] = jax.nn.gelu(
                    ccw_bufs[rcv, :, :] + t_ccw, approximate=True)

    return pl.pallas_call(
        body,
        out_shape=jax.ShapeDtypeStruct((m_per, n), jnp.float32),
        in_specs=[
            pl.BlockSpec(memory_space=pltpu.VMEM),
            pl.BlockSpec(memory_space=pltpu.VMEM),
        ],
        out_specs=pl.BlockSpec(memory_space=pltpu.VMEM),
        scratch_shapes=[
            pltpu.VMEM((2, m_per, n_half), jnp.float32),
            pltpu.VMEM((2, m_per, n_half), jnp.float32),
            pltpu.SemaphoreType.DMA((N_DEV - 1,)),
            pltpu.SemaphoreType.DMA((N_DEV - 1,)),
            pltpu.SemaphoreType.DMA((N_DEV - 1,)),
            pltpu.SemaphoreType.DMA((N_DEV - 1,)),
        ],
        compiler_params=pltpu.CompilerParams(
            collective_id=0,
            vmem_limit_bytes=100 * 1024 * 1024,
        ),
    )(x, w_mat)


# device time: 161466 ns/iter; 1.9665x vs baseline; 1.0485x over previous
import jax
import jax.numpy as jnp
from jax import lax
from jax.experimental import pallas as pl
from jax.experimental.pallas import tpu as pltpu

N_DEV = 4
SUB = 2


def kernel(x, w_mat):
    m_glob, k_per = x.shape
    _, n = w_mat.shape
    m_per = m_glob // N_DEV
    n_half = n // 2
    m_sub = m_per // SUB

    def body(x_ref, w_ref, out_ref,
             cw_bufs, ccw_bufs,
             cw_send_sems, cw_recv_sems, ccw_send_sems, ccw_recv_sems):
        p = lax.axis_index("i")
        left = lax.rem(p + N_DEV - 1, N_DEV)
        right = lax.rem(p + 1, N_DEV)

        barrier_sem = pltpu.get_barrier_semaphore()
        for nbr in (left, right):
            pl.semaphore_signal(
                barrier_sem, inc=1,
                device_id=(nbr,), device_id_type=pl.DeviceIdType.MESH,
            )
        pl.semaphore_wait(barrier_sem, 2)

        def partial(c, half, j):
            xs = x_ref[pl.ds(c * m_per + j * m_sub, m_sub), :]
            ws = w_ref[:, half * n_half:(half + 1) * n_half]
            return jnp.dot(xs, ws, preferred_element_type=jnp.float32)

        def make_rdma(bufs, s, j, sems_pair, target):
            send_sems, recv_sems = sems_pair
            rows = pl.ds(j * m_sub, m_sub)
            return pltpu.make_async_remote_copy(
                src_ref=bufs.at[s % 2, rows],
                dst_ref=bufs.at[(s + 1) % 2, rows],
                send_sem=send_sems.at[s, j],
                recv_sem=recv_sems.at[s, j],
                device_id=(target,),
                device_id_type=pl.DeviceIdType.MESH,
            )

        cw_sems = (cw_send_sems, cw_recv_sems)
        ccw_sems = (ccw_send_sems, ccw_recv_sems)

        c_cw0 = lax.rem(p + N_DEV - 1, N_DEV)
        c_ccw0 = lax.rem(p + 1, N_DEV)
        seed_rdmas = []
        for j in range(SUB):
            rows = pl.ds(j * m_sub, m_sub)
            cw_bufs[0, rows, :] = partial(c_cw0, 0, j)
            r = make_rdma(cw_bufs, 0, j, cw_sems, right)
            r.start()
            seed_rdmas.append(r)
            ccw_bufs[0, rows, :] = partial(c_ccw0, 1, j)
            r = make_rdma(ccw_bufs, 0, j, ccw_sems, left)
            r.start()
            seed_rdmas.append(r)

        for s in range(N_DEV - 1):
            rcv = (s + 1) % 2
            c_cw = lax.rem(p + 2 * N_DEV - 2 - s, N_DEV)
            c_ccw = lax.rem(p + 2 + s, N_DEV)
            for j in range(SUB):
                rows = pl.ds(j * m_sub, m_sub)
                t_cw = partial(c_cw, 0, j)
                t_ccw = partial(c_ccw, 1, j)

                make_rdma(cw_bufs, s, j, cw_sems, right).wait_recv()
                if s < N_DEV - 2:
                    cw_bufs[rcv, rows, :] = cw_bufs[rcv, rows, :] + t_cw
                    make_rdma(cw_bufs, s + 1, j, cw_sems, right).start()
                else:
                    out_ref[rows, :n_half] = jax.nn.gelu(
                        cw_bufs[rcv, rows, :] + t_cw, approximate=True)

                make_rdma(ccw_bufs, s, j, ccw_sems, left).wait_recv()
                if s < N_DEV - 2:
                    ccw_bufs[rcv, rows, :] = ccw_bufs[rcv, rows, :] + t_ccw
                    make_rdma(ccw_bufs, s + 1, j, ccw_sems, left).start()
                else:
                    out_ref[rows, n_half:] = jax.nn.gelu(
                        ccw_bufs[rcv, rows, :] + t_ccw, approximate=True)

        for r in seed_rdmas:
            r.wait_send()
        for s in range(1, N_DEV - 1):
            for j in range(SUB):
                make_rdma(cw_bufs, s, j, cw_sems, right).wait_send()
                make_rdma(ccw_bufs, s, j, ccw_sems, left).wait_send()

    return pl.pallas_call(
        body,
        out_shape=jax.ShapeDtypeStruct((m_per, n), jnp.float32),
        in_specs=[
            pl.BlockSpec(memory_space=pltpu.VMEM),
            pl.BlockSpec(memory_space=pltpu.VMEM),
        ],
        out_specs=pl.BlockSpec(memory_space=pltpu.VMEM),
        scratch_shapes=[
            pltpu.VMEM((2, m_per, n_half), jnp.float32),
            pltpu.VMEM((2, m_per, n_half), jnp.float32),
            pltpu.SemaphoreType.DMA((N_DEV - 1, SUB)),
            pltpu.SemaphoreType.DMA((N_DEV - 1, SUB)),
            pltpu.SemaphoreType.DMA((N_DEV - 1, SUB)),
            pltpu.SemaphoreType.DMA((N_DEV - 1, SUB)),
        ],
        compiler_params=pltpu.CompilerParams(
            collective_id=0,
            vmem_limit_bytes=100 * 1024 * 1024,
        ),
    )(x, w_mat)


# device time: 160072 ns/iter; 1.9836x vs baseline; 1.0087x over previous
import jax
import jax.numpy as jnp
from jax import lax
from jax.experimental import pallas as pl
from jax.experimental.pallas import tpu as pltpu

N_DEV = 4
SUB = 4


def kernel(x, w_mat):
    m_glob, k_per = x.shape
    _, n = w_mat.shape
    m_per = m_glob // N_DEV
    n_half = n // 2
    m_sub = m_per // SUB

    def body(x_ref, w_ref, out_ref,
             cw_bufs, ccw_bufs,
             cw_send_sems, cw_recv_sems, ccw_send_sems, ccw_recv_sems):
        p = lax.axis_index("i")
        left = lax.rem(p + N_DEV - 1, N_DEV)
        right = lax.rem(p + 1, N_DEV)

        barrier_sem = pltpu.get_barrier_semaphore()
        for nbr in (left, right):
            pl.semaphore_signal(
                barrier_sem, inc=1,
                device_id=(nbr,), device_id_type=pl.DeviceIdType.MESH,
            )
        pl.semaphore_wait(barrier_sem, 2)

        def partial(c, half, j):
            xs = x_ref[pl.ds(c * m_per + j * m_sub, m_sub), :]
            ws = w_ref[:, half * n_half:(half + 1) * n_half]
            return jnp.dot(xs, ws, preferred_element_type=jnp.float32)

        def make_rdma(bufs, s, j, sems_pair, target):
            send_sems, recv_sems = sems_pair
            rows = pl.ds(j * m_sub, m_sub)
            return pltpu.make_async_remote_copy(
                src_ref=bufs.at[s % 2, rows],
                dst_ref=bufs.at[(s + 1) % 2, rows],
                send_sem=send_sems.at[s, j],
                recv_sem=recv_sems.at[s, j],
                device_id=(target,),
                device_id_type=pl.DeviceIdType.MESH,
            )

        cw_sems = (cw_send_sems, cw_recv_sems)
        ccw_sems = (ccw_send_sems, ccw_recv_sems)

        c_cw0 = lax.rem(p + N_DEV - 1, N_DEV)
        c_ccw0 = lax.rem(p + 1, N_DEV)
        seed_rdmas = []
        for j in range(SUB):
            rows = pl.ds(j * m_sub, m_sub)
            cw_bufs[0, rows, :] = partial(c_cw0, 0, j)
            r = make_rdma(cw_bufs, 0, j, cw_sems, right)
            r.start()
            seed_rdmas.append(r)
            ccw_bufs[0, rows, :] = partial(c_ccw0, 1, j)
            r = make_rdma(ccw_bufs, 0, j, ccw_sems, left)
            r.start()
            seed_rdmas.append(r)

        for s in range(N_DEV - 1):
            rcv = (s + 1) % 2
            c_cw = lax.rem(p + 2 * N_DEV - 2 - s, N_DEV)
            c_ccw = lax.rem(p + 2 + s, N_DEV)
            for j in range(SUB):
                rows = pl.ds(j * m_sub, m_sub)
                t_cw = partial(c_cw, 0, j)
                t_ccw = partial(c_ccw, 1, j)

                make_rdma(cw_bufs, s, j, cw_sems, right).wait_recv()
                if s < N_DEV - 2:
                    cw_bufs[rcv, rows, :] = cw_bufs[rcv, rows, :] + t_cw
                    make_rdma(cw_bufs, s + 1, j, cw_sems, right).start()
                else:
                    out_ref[rows, :n_half] = jax.nn.gelu(
                        cw_bufs[rcv, rows, :] + t_cw, approximate=True)

                make_rdma(ccw_bufs, s, j, ccw_sems, left).wait_recv()
                if s < N_DEV - 2:
                    ccw_bufs[rcv, rows, :] = ccw_bufs[rcv, rows, :] + t_ccw
                    make_rdma(ccw_bufs, s + 1, j, ccw_sems, left).start()
                else:
                    out_ref[rows, n_half:] = jax.nn.gelu(
                        ccw_bufs[rcv, rows, :] + t_ccw, approximate=True)

        for r in seed_rdmas:
            r.wait_send()
        for s in range(1, N_DEV - 1):
            for j in range(SUB):
                make_rdma(cw_bufs, s, j, cw_sems, right).wait_send()
                make_rdma(ccw_bufs, s, j, ccw_sems, left).wait_send()

    return pl.pallas_call(
        body,
        out_shape=jax.ShapeDtypeStruct((m_per, n), jnp.float32),
        in_specs=[
            pl.BlockSpec(memory_space=pltpu.VMEM),
            pl.BlockSpec(memory_space=pltpu.VMEM),
        ],
        out_specs=pl.BlockSpec(memory_space=pltpu.VMEM),
        scratch_shapes=[
            pltpu.VMEM((2, m_per, n_half), jnp.float32),
            pltpu.VMEM((2, m_per, n_half), jnp.float32),
            pltpu.SemaphoreType.DMA((N_DEV - 1, SUB)),
            pltpu.SemaphoreType.DMA((N_DEV - 1, SUB)),
            pltpu.SemaphoreType.DMA((N_DEV - 1, SUB)),
            pltpu.SemaphoreType.DMA((N_DEV - 1, SUB)),
        ],
        compiler_params=pltpu.CompilerParams(
            collective_id=0,
            vmem_limit_bytes=100 * 1024 * 1024,
        ),
    )(x, w_mat)
